# baseline (speedup 1.0000x reference)
import functools

import jax
import jax.numpy as jnp
from jax import lax
from jax.experimental import pallas as pl
from jax.experimental.pallas import tpu as pltpu
from jax.experimental.pallas import tpu_sc as plsc

_NC = 2
_NS = 16
_NW = _NC * _NS
_L = 16

_T = 100
_T_PAD = 112
_TK = 4
_IN_SIZE = 512.0

_mesh = plsc.VectorSubcoreMesh(core_axis_name="c", subcore_axis_name="s")


def _make_sc_kernel(b, a_dim, h_dim, w_dim):
    hpw = h_dim // (_NW // b)
    rows = a_dim * hpw
    npw = rows * w_dim
    div = _NW // b

    @functools.partial(
        pl.kernel,
        out_type=jax.ShapeDtypeStruct((b * a_dim, h_dim, w_dim), jnp.float32),
        mesh=_mesh,
        compiler_params=pltpu.CompilerParams(needs_layout_passes=False,
                                             use_tc_tiling_on_sc=False),
        scratch_types=[
            pltpu.VMEM((rows, w_dim * 4), jnp.float32),
            pltpu.VMEM((_T_PAD * 4,), jnp.float32),
            pltpu.VMEM((rows, w_dim), jnp.float32),
            pltpu.VMEM((rows, w_dim), jnp.float32),
            pltpu.VMEM((npw,), jnp.float32),
            pltpu.VMEM((npw,), jnp.float32),
            pltpu.VMEM((npw,), jnp.float32),
            pltpu.VMEM((npw,), jnp.float32),
            pltpu.VMEM((_T_PAD,), jnp.float32),
            pltpu.VMEM((_T_PAD,), jnp.float32),
            pltpu.VMEM((_T_PAD,), jnp.float32),
            pltpu.VMEM((_T_PAD,), jnp.float32),
            pltpu.VMEM((_T_PAD,), jnp.float32),
            pltpu.VMEM((npw,), jnp.float32),
        ],
    )
    def sc_kernel(pred_hbm, tgt_hbm, mask_hbm, out_hbm,
                  praw_v, traw_v, mrow_v, orow_v,
                  px1_v, py1_v, px2_v, py2_v,
                  tx1_v, ty1_v, tx2_v, ty2_v, ta3_v, macc_v):
        wid = lax.axis_index("s") * _NC + lax.axis_index("c")
        img = wid // div
        h0 = (wid % div) * hpw

        for a in range(a_dim):
            pltpu.sync_copy(pred_hbm.at[img * a_dim + a, pl.ds(h0, hpw)],
                            praw_v.at[pl.ds(a * hpw, hpw)])
        pltpu.sync_copy(tgt_hbm.at[img], traw_v)

        lane = lax.iota(jnp.int32, _L)
        lane4 = lane * 4
        cpr = w_dim // _L

        init = (jnp.full((_L,), -3.4e38, jnp.float32),
                jnp.full((_L,), 3.4e38, jnp.float32))

        @plsc.parallel_loop(0, rows, 1, unroll=2, carry=init)
        def pscan(r, ext):
            amax, amin = ext
            rr = jnp.full((_L,), r, jnp.int32)
            for k in range(cpr):
                base = lane4 + (k * _L * 4)
                w = plsc.load_gather(praw_v, [rr, base + 2])
                h = plsc.load_gather(praw_v, [rr, base + 3])
                pa = w * h
                amax = jnp.maximum(amax, pa)
                amin = jnp.minimum(amin, pa)
            return amax, amin

        pamax3 = jnp.max(pscan[0]) * (1.0001 / 3.0)
        pamin3 = jnp.min(pscan[1]) * (0.9999 / 3.0)

        alive_acc = jnp.zeros((_L,), jnp.int32)
        for j in range(_T_PAD // _L):
            base = lane4 + (j * _L * 4)
            tw = plsc.load_gather(traw_v, [base + 2]) * _IN_SIZE
            th = plsc.load_gather(traw_v, [base + 3]) * _IN_SIZE
            ta3 = tw * th * (1.0 / 3.0)
            live = jnp.logical_and(ta3 < pamax3 * 6.0, ta3 * 6.0 > pamin3)
            alive_acc = alive_acc | jnp.where(live, 1, 0)
        any_alive = jnp.max(alive_acc)

        @pl.when(any_alive == 0)
        def _():
            for a in range(a_dim):
                pltpu.sync_copy(mask_hbm.at[img * a_dim + a, pl.ds(h0, hpw)],
                                mrow_v.at[pl.ds(a * hpw, hpw)])
            for a in range(a_dim):
                pltpu.sync_copy(mrow_v.at[pl.ds(a * hpw, hpw)],
                                out_hbm.at[img * a_dim + a, pl.ds(h0, hpw)])

        @pl.when(any_alive > 0)
        def _():
            for a in range(a_dim):
                pltpu.sync_copy(mask_hbm.at[img * a_dim + a, pl.ds(h0, hpw)],
                                mrow_v.at[pl.ds(a * hpw, hpw)])

            for j in range(_T_PAD // _L):
                base = lane4 + (j * _L * 4)
                cx = plsc.load_gather(traw_v, [base]) * _IN_SIZE
                cy = plsc.load_gather(traw_v, [base + 1]) * _IN_SIZE
                hw = plsc.load_gather(traw_v, [base + 2]) * (0.5 * _IN_SIZE)
                hh = plsc.load_gather(traw_v, [base + 3]) * (0.5 * _IN_SIZE)
                sl = pl.ds(j * _L, _L)
                x1 = cx - hw
                y1 = cy - hh
                x2 = cx + hw
                y2 = cy + hh
                tx1_v[sl] = x1
                ty1_v[sl] = y1
                tx2_v[sl] = x2
                ty2_v[sl] = y2
                ta3_v[sl] = (x2 - x1) * (y2 - y1) * (1.0 / 3.0)

            @plsc.parallel_loop(0, rows, 1, unroll=2)
            def pprep(r):
                rr = jnp.full((_L,), r, jnp.int32)
                for k in range(cpr):
                    base = lane4 + (k * _L * 4)
                    cx = plsc.load_gather(praw_v, [rr, base])
                    cy = plsc.load_gather(praw_v, [rr, base + 1])
                    hw = plsc.load_gather(praw_v, [rr, base + 2]) * 0.5
                    hh = plsc.load_gather(praw_v, [rr, base + 3]) * 0.5
                    sl = pl.ds(r * w_dim + k * _L, _L)
                    px1_v[sl] = cx - hw
                    py1_v[sl] = cy - hh
                    px2_v[sl] = cx + hw
                    py2_v[sl] = cy + hh
                    macc_v[sl] = jnp.zeros((_L,), jnp.float32)

            def tblk(tb, carry):
                t0 = tb * _TK
                bts = []
                for k in range(_TK):
                    idx = jnp.full((_L,), t0 + k, jnp.int32)
                    bts.append((plsc.load_gather(tx1_v, [idx]),
                                plsc.load_gather(ty1_v, [idx]),
                                plsc.load_gather(tx2_v, [idx]),
                                plsc.load_gather(ty2_v, [idx]),
                                plsc.load_gather(ta3_v, [idx])))

                blk_alive = jnp.zeros((_L,), jnp.int32)
                for (_, _, _, _, bta3) in bts:
                    live = jnp.logical_and(bta3 < pamax3 * 6.0,
                                           bta3 * 6.0 > pamin3)
                    blk_alive = blk_alive | jnp.where(live, 1, 0)

                @pl.when(jnp.max(blk_alive) > 0)
                def _():
                    @plsc.parallel_loop(0, npw // _L, 1, unroll=4)
                    def ploop(p):
                        sl = pl.ds(p * _L, _L)
                        px1 = px1_v[sl]
                        py1 = py1_v[sl]
                        px2 = px2_v[sl]
                        py2 = py2_v[sl]
                        m = macc_v[sl]
                        for (btx1, bty1, btx2, bty2, bta3) in bts:
                            iw = jnp.maximum(
                                jnp.minimum(btx2, px2) - jnp.maximum(btx1, px1),
                                0.0)
                            ih = jnp.maximum(
                                jnp.minimum(bty2, py2) - jnp.maximum(bty1, py1),
                                0.0)
                            m = jnp.maximum(m, iw * ih - bta3)
                        macc_v[sl] = m

                return carry

            lax.fori_loop(0, _T // _TK, tblk, 0)

            @plsc.parallel_loop(0, rows, 1, unroll=2)
            def fin(r):
                for k in range(cpr):
                    sl = pl.ds(r * w_dim + k * _L, _L)
                    ms = pl.ds(k * _L, _L)
                    pa3 = ((px2_v[sl] - px1_v[sl])
                           * (py2_v[sl] - py1_v[sl]) * (1.0 / 3.0))
                    orow_v[r, ms] = jnp.where(
                        macc_v[sl] > pa3, 0.0, mrow_v[r, ms])

            for a in range(a_dim):
                pltpu.sync_copy(orow_v.at[pl.ds(a * hpw, hpw)],
                                out_hbm.at[img * a_dim + a, pl.ds(h0, hpw)])

    return sc_kernel


def kernel(batch_predict_boxes, batch_targets, no_obj_mask):
    b, a_dim, h_dim, w_dim, _ = batch_predict_boxes.shape
    pred = batch_predict_boxes.reshape(b * a_dim, h_dim, w_dim * 4)
    mask = no_obj_mask.reshape(b * a_dim, h_dim, w_dim)
    tgt = jnp.pad(batch_targets, ((0, 0), (0, _T_PAD - _T), (0, 0)))
    tgt = tgt.reshape(b, _T_PAD * 4)
    out = _make_sc_kernel(b, a_dim, h_dim, w_dim)(pred, tgt, mask)
    return out.reshape(no_obj_mask.shape)


# speedup vs baseline: 1.0803x; 1.0803x over previous
import functools

import jax
import jax.numpy as jnp
from jax import lax
from jax.experimental import pallas as pl
from jax.experimental.pallas import tpu as pltpu
from jax.experimental.pallas import tpu_sc as plsc

_NC = 2
_NS = 16
_NW = _NC * _NS
_L = 16

_T = 100
_T_PAD = 112
_TK = 4
_IN_SIZE = 512.0

_mesh = plsc.VectorSubcoreMesh(core_axis_name="c", subcore_axis_name="s")


def _make_sc_kernel(b, a_dim, h_dim, w_dim):
    hpw = h_dim // (_NW // b)
    rows = a_dim * hpw
    npw = rows * w_dim
    div = _NW // b

    @functools.partial(
        pl.kernel,
        out_type=jax.ShapeDtypeStruct((b * a_dim, h_dim, w_dim), jnp.float32),
        mesh=_mesh,
        compiler_params=pltpu.CompilerParams(needs_layout_passes=False),
        scratch_types=[
            pltpu.VMEM((rows, w_dim * 4), jnp.float32),
            pltpu.VMEM((_T_PAD * 4,), jnp.float32),
            pltpu.VMEM((rows, w_dim), jnp.float32),
            pltpu.VMEM((rows, w_dim), jnp.float32),
            pltpu.VMEM((npw,), jnp.float32),
            pltpu.VMEM((npw,), jnp.float32),
            pltpu.VMEM((npw,), jnp.float32),
            pltpu.VMEM((npw,), jnp.float32),
            pltpu.VMEM((_T_PAD,), jnp.float32),
            pltpu.VMEM((_T_PAD,), jnp.float32),
            pltpu.VMEM((_T_PAD,), jnp.float32),
            pltpu.VMEM((_T_PAD,), jnp.float32),
            pltpu.VMEM((_T_PAD,), jnp.float32),
            pltpu.VMEM((npw,), jnp.float32),
        ],
    )
    def sc_kernel(pred_hbm, tgt_hbm, mask_hbm, out_hbm,
                  praw_v, traw_v, mrow_v, orow_v,
                  px1_v, py1_v, px2_v, py2_v,
                  tx1_v, ty1_v, tx2_v, ty2_v, ta3_v, macc_v):
        wid = lax.axis_index("s") * _NC + lax.axis_index("c")
        img = wid // div
        h0 = (wid % div) * hpw

        for a in range(a_dim):
            pltpu.sync_copy(pred_hbm.at[img * a_dim + a, pl.ds(h0, hpw)],
                            praw_v.at[pl.ds(a * hpw, hpw)])
        pltpu.sync_copy(tgt_hbm.at[img], traw_v)

        lane = lax.iota(jnp.int32, _L)
        lane4 = lane * 4
        cpr = w_dim // _L

        init = (jnp.full((_L,), -3.4e38, jnp.float32),
                jnp.full((_L,), 3.4e38, jnp.float32))

        @plsc.parallel_loop(0, rows, 1, unroll=2, carry=init)
        def pscan(r, ext):
            amax, amin = ext
            for k in range(cpr * 4):
                v = praw_v[r, pl.ds(k * _L, _L)]
                amax = jnp.maximum(amax, v)
                amin = jnp.minimum(amin, v)
            return amax, amin

        is_w = (lane % 4) == 2
        is_h = (lane % 4) == 3
        wmax = jnp.max(jnp.where(is_w, pscan[0], -3.4e38))
        hmax = jnp.max(jnp.where(is_h, pscan[0], -3.4e38))
        wmin = jnp.min(jnp.where(is_w, pscan[1], 3.4e38))
        hmin = jnp.min(jnp.where(is_h, pscan[1], 3.4e38))
        pamax3 = wmax * hmax * (1.0001 / 3.0)
        pamin3 = wmin * hmin * (0.9999 / 3.0)

        alive_acc = jnp.zeros((_L,), jnp.int32)
        for j in range(_T_PAD // _L):
            base = lane4 + (j * _L * 4)
            tw = plsc.load_gather(traw_v, [base + 2]) * _IN_SIZE
            th = plsc.load_gather(traw_v, [base + 3]) * _IN_SIZE
            ta3 = tw * th * (1.0 / 3.0)
            live = jnp.logical_and(ta3 < pamax3 * 6.0, ta3 * 6.0 > pamin3)
            alive_acc = alive_acc | jnp.where(live, 1, 0)
        any_alive = jnp.max(alive_acc)

        @pl.when(any_alive == 0)
        def _():
            for a in range(a_dim):
                pltpu.sync_copy(mask_hbm.at[img * a_dim + a, pl.ds(h0, hpw)],
                                mrow_v.at[pl.ds(a * hpw, hpw)])
            for a in range(a_dim):
                pltpu.sync_copy(mrow_v.at[pl.ds(a * hpw, hpw)],
                                out_hbm.at[img * a_dim + a, pl.ds(h0, hpw)])

        @pl.when(any_alive > 0)
        def _():
            for a in range(a_dim):
                pltpu.sync_copy(mask_hbm.at[img * a_dim + a, pl.ds(h0, hpw)],
                                mrow_v.at[pl.ds(a * hpw, hpw)])

            for j in range(_T_PAD // _L):
                base = lane4 + (j * _L * 4)
                cx = plsc.load_gather(traw_v, [base]) * _IN_SIZE
                cy = plsc.load_gather(traw_v, [base + 1]) * _IN_SIZE
                hw = plsc.load_gather(traw_v, [base + 2]) * (0.5 * _IN_SIZE)
                hh = plsc.load_gather(traw_v, [base + 3]) * (0.5 * _IN_SIZE)
                sl = pl.ds(j * _L, _L)
                x1 = cx - hw
                y1 = cy - hh
                x2 = cx + hw
                y2 = cy + hh
                tx1_v[sl] = x1
                ty1_v[sl] = y1
                tx2_v[sl] = x2
                ty2_v[sl] = y2
                ta3_v[sl] = (x2 - x1) * (y2 - y1) * (1.0 / 3.0)

            @plsc.parallel_loop(0, rows, 1, unroll=2)
            def pprep(r):
                rr = jnp.full((_L,), r, jnp.int32)
                for k in range(cpr):
                    base = lane4 + (k * _L * 4)
                    cx = plsc.load_gather(praw_v, [rr, base])
                    cy = plsc.load_gather(praw_v, [rr, base + 1])
                    hw = plsc.load_gather(praw_v, [rr, base + 2]) * 0.5
                    hh = plsc.load_gather(praw_v, [rr, base + 3]) * 0.5
                    sl = pl.ds(r * w_dim + k * _L, _L)
                    px1_v[sl] = cx - hw
                    py1_v[sl] = cy - hh
                    px2_v[sl] = cx + hw
                    py2_v[sl] = cy + hh
                    macc_v[sl] = jnp.zeros((_L,), jnp.float32)

            def tblk(tb, carry):
                t0 = tb * _TK
                bts = []
                for k in range(_TK):
                    idx = jnp.full((_L,), t0 + k, jnp.int32)
                    bts.append((plsc.load_gather(tx1_v, [idx]),
                                plsc.load_gather(ty1_v, [idx]),
                                plsc.load_gather(tx2_v, [idx]),
                                plsc.load_gather(ty2_v, [idx]),
                                plsc.load_gather(ta3_v, [idx])))

                blk_alive = jnp.zeros((_L,), jnp.int32)
                for (_, _, _, _, bta3) in bts:
                    live = jnp.logical_and(bta3 < pamax3 * 6.0,
                                           bta3 * 6.0 > pamin3)
                    blk_alive = blk_alive | jnp.where(live, 1, 0)

                @pl.when(jnp.max(blk_alive) > 0)
                def _():
                    @plsc.parallel_loop(0, npw // _L, 1, unroll=4)
                    def ploop(p):
                        sl = pl.ds(p * _L, _L)
                        px1 = px1_v[sl]
                        py1 = py1_v[sl]
                        px2 = px2_v[sl]
                        py2 = py2_v[sl]
                        m = macc_v[sl]
                        for (btx1, bty1, btx2, bty2, bta3) in bts:
                            iw = jnp.maximum(
                                jnp.minimum(btx2, px2) - jnp.maximum(btx1, px1),
                                0.0)
                            ih = jnp.maximum(
                                jnp.minimum(bty2, py2) - jnp.maximum(bty1, py1),
                                0.0)
                            m = jnp.maximum(m, iw * ih - bta3)
                        macc_v[sl] = m

                return carry

            lax.fori_loop(0, _T // _TK, tblk, 0)

            @plsc.parallel_loop(0, rows, 1, unroll=2)
            def fin(r):
                for k in range(cpr):
                    sl = pl.ds(r * w_dim + k * _L, _L)
                    ms = pl.ds(k * _L, _L)
                    pa3 = ((px2_v[sl] - px1_v[sl])
                           * (py2_v[sl] - py1_v[sl]) * (1.0 / 3.0))
                    orow_v[r, ms] = jnp.where(
                        macc_v[sl] > pa3, 0.0, mrow_v[r, ms])

            for a in range(a_dim):
                pltpu.sync_copy(orow_v.at[pl.ds(a * hpw, hpw)],
                                out_hbm.at[img * a_dim + a, pl.ds(h0, hpw)])

    return sc_kernel


def kernel(batch_predict_boxes, batch_targets, no_obj_mask):
    b, a_dim, h_dim, w_dim, _ = batch_predict_boxes.shape
    pred = batch_predict_boxes.reshape(b * a_dim, h_dim, w_dim * 4)
    mask = no_obj_mask.reshape(b * a_dim, h_dim, w_dim)
    tgt = jnp.pad(batch_targets, ((0, 0), (0, _T_PAD - _T), (0, 0)))
    tgt = tgt.reshape(b, _T_PAD * 4)
    out = _make_sc_kernel(b, a_dim, h_dim, w_dim)(pred, tgt, mask)
    return out.reshape(no_obj_mask.shape)


# speedup vs baseline: 1.1678x; 1.0810x over previous
import functools

import jax
import jax.numpy as jnp
from jax import lax
from jax.experimental import pallas as pl
from jax.experimental.pallas import tpu as pltpu
from jax.experimental.pallas import tpu_sc as plsc

_NC = 2
_NS = 16
_NW = _NC * _NS
_L = 16

_T = 100
_T_PAD = 112
_TK = 4
_IN_SIZE = 512.0

_mesh = plsc.VectorSubcoreMesh(core_axis_name="c", subcore_axis_name="s")


def _make_sc_kernel(b, a_dim, h_dim, w_dim):
    hpw = h_dim // (_NW // b)
    rows = a_dim * hpw
    npw = rows * w_dim
    div = _NW // b

    @functools.partial(
        pl.kernel,
        out_type=jax.ShapeDtypeStruct((b * a_dim, h_dim, w_dim), jnp.float32),
        mesh=_mesh,
        compiler_params=pltpu.CompilerParams(needs_layout_passes=False),
        scratch_types=[
            pltpu.VMEM((rows, w_dim * 4), jnp.float32),
            pltpu.VMEM((_T_PAD * 4,), jnp.float32),
            pltpu.VMEM((rows, w_dim), jnp.float32),
            pltpu.VMEM((rows, w_dim), jnp.float32),
            pltpu.VMEM((npw,), jnp.float32),
            pltpu.VMEM((npw,), jnp.float32),
            pltpu.VMEM((npw,), jnp.float32),
            pltpu.VMEM((npw,), jnp.float32),
            pltpu.VMEM((_T_PAD,), jnp.float32),
            pltpu.VMEM((_T_PAD,), jnp.float32),
            pltpu.VMEM((_T_PAD,), jnp.float32),
            pltpu.VMEM((_T_PAD,), jnp.float32),
            pltpu.VMEM((_T_PAD,), jnp.float32),
            pltpu.VMEM((npw,), jnp.float32),
            pltpu.SemaphoreType.DMA,
            pltpu.SemaphoreType.DMA,
        ],
    )
    def sc_kernel(pred_hbm, tgt_hbm, mask_hbm, out_hbm,
                  praw_v, traw_v, mrow_v, orow_v,
                  px1_v, py1_v, px2_v, py2_v,
                  tx1_v, ty1_v, tx2_v, ty2_v, ta3_v, macc_v,
                  sem_in, sem_mask):
        wid = lax.axis_index("s") * _NC + lax.axis_index("c")
        img = wid // div
        h0 = (wid % div) * hpw

        in_copies = [
            pltpu.async_copy(pred_hbm.at[img * a_dim + a, pl.ds(h0, hpw)],
                             praw_v.at[pl.ds(a * hpw, hpw)], sem_in)
            for a in range(a_dim)
        ]
        in_copies.append(pltpu.async_copy(tgt_hbm.at[img], traw_v, sem_in))
        mask_copies = [
            pltpu.async_copy(mask_hbm.at[img * a_dim + a, pl.ds(h0, hpw)],
                             mrow_v.at[pl.ds(a * hpw, hpw)], sem_mask)
            for a in range(a_dim)
        ]
        for c in in_copies:
            c.wait()

        lane = lax.iota(jnp.int32, _L)
        lane4 = lane * 4
        cpr = w_dim // _L

        init = (jnp.full((_L,), -3.4e38, jnp.float32),
                jnp.full((_L,), 3.4e38, jnp.float32))

        @plsc.parallel_loop(0, rows, 1, unroll=2, carry=init)
        def pscan(r, ext):
            amax, amin = ext
            for k in range(cpr * 4):
                v = praw_v[r, pl.ds(k * _L, _L)]
                amax = jnp.maximum(amax, v)
                amin = jnp.minimum(amin, v)
            return amax, amin

        is_w = (lane % 4) == 2
        is_h = (lane % 4) == 3
        wmax = jnp.max(jnp.where(is_w, pscan[0], -3.4e38))
        hmax = jnp.max(jnp.where(is_h, pscan[0], -3.4e38))
        wmin = jnp.min(jnp.where(is_w, pscan[1], 3.4e38))
        hmin = jnp.min(jnp.where(is_h, pscan[1], 3.4e38))
        pamax3 = wmax * hmax * (1.0001 / 3.0)
        pamin3 = wmin * hmin * (0.9999 / 3.0)

        alive_acc = jnp.zeros((_L,), jnp.int32)
        for j in range(_T_PAD // _L):
            base = lane4 + (j * _L * 4)
            tw = plsc.load_gather(traw_v, [base + 2]) * _IN_SIZE
            th = plsc.load_gather(traw_v, [base + 3]) * _IN_SIZE
            ta3 = tw * th * (1.0 / 3.0)
            live = jnp.logical_and(ta3 < pamax3 * 6.0, ta3 * 6.0 > pamin3)
            alive_acc = alive_acc | jnp.where(live, 1, 0)
        any_alive = jnp.max(alive_acc)

        @pl.when(any_alive == 0)
        def _():
            for c in mask_copies:
                c.wait()
            for a in range(a_dim):
                pltpu.sync_copy(mrow_v.at[pl.ds(a * hpw, hpw)],
                                out_hbm.at[img * a_dim + a, pl.ds(h0, hpw)])

        @pl.when(any_alive > 0)
        def _():
            for c in mask_copies:
                c.wait()

            for j in range(_T_PAD // _L):
                base = lane4 + (j * _L * 4)
                cx = plsc.load_gather(traw_v, [base]) * _IN_SIZE
                cy = plsc.load_gather(traw_v, [base + 1]) * _IN_SIZE
                hw = plsc.load_gather(traw_v, [base + 2]) * (0.5 * _IN_SIZE)
                hh = plsc.load_gather(traw_v, [base + 3]) * (0.5 * _IN_SIZE)
                sl = pl.ds(j * _L, _L)
                x1 = cx - hw
                y1 = cy - hh
                x2 = cx + hw
                y2 = cy + hh
                tx1_v[sl] = x1
                ty1_v[sl] = y1
                tx2_v[sl] = x2
                ty2_v[sl] = y2
                ta3_v[sl] = (x2 - x1) * (y2 - y1) * (1.0 / 3.0)

            @plsc.parallel_loop(0, rows, 1, unroll=2)
            def pprep(r):
                rr = jnp.full((_L,), r, jnp.int32)
                for k in range(cpr):
                    base = lane4 + (k * _L * 4)
                    cx = plsc.load_gather(praw_v, [rr, base])
                    cy = plsc.load_gather(praw_v, [rr, base + 1])
                    hw = plsc.load_gather(praw_v, [rr, base + 2]) * 0.5
                    hh = plsc.load_gather(praw_v, [rr, base + 3]) * 0.5
                    sl = pl.ds(r * w_dim + k * _L, _L)
                    px1_v[sl] = cx - hw
                    py1_v[sl] = cy - hh
                    px2_v[sl] = cx + hw
                    py2_v[sl] = cy + hh
                    macc_v[sl] = jnp.zeros((_L,), jnp.float32)

            def tblk(tb, carry):
                t0 = tb * _TK
                bts = []
                for k in range(_TK):
                    idx = jnp.full((_L,), t0 + k, jnp.int32)
                    bts.append((plsc.load_gather(tx1_v, [idx]),
                                plsc.load_gather(ty1_v, [idx]),
                                plsc.load_gather(tx2_v, [idx]),
                                plsc.load_gather(ty2_v, [idx]),
                                plsc.load_gather(ta3_v, [idx])))

                blk_alive = jnp.zeros((_L,), jnp.int32)
                for (_, _, _, _, bta3) in bts:
                    live = jnp.logical_and(bta3 < pamax3 * 6.0,
                                           bta3 * 6.0 > pamin3)
                    blk_alive = blk_alive | jnp.where(live, 1, 0)

                @pl.when(jnp.max(blk_alive) > 0)
                def _():
                    @plsc.parallel_loop(0, npw // _L, 1, unroll=4)
                    def ploop(p):
                        sl = pl.ds(p * _L, _L)
                        px1 = px1_v[sl]
                        py1 = py1_v[sl]
                        px2 = px2_v[sl]
                        py2 = py2_v[sl]
                        m = macc_v[sl]
                        for (btx1, bty1, btx2, bty2, bta3) in bts:
                            iw = jnp.maximum(
                                jnp.minimum(btx2, px2) - jnp.maximum(btx1, px1),
                                0.0)
                            ih = jnp.maximum(
                                jnp.minimum(bty2, py2) - jnp.maximum(bty1, py1),
                                0.0)
                            m = jnp.maximum(m, iw * ih - bta3)
                        macc_v[sl] = m

                return carry

            lax.fori_loop(0, _T // _TK, tblk, 0)

            @plsc.parallel_loop(0, rows, 1, unroll=2)
            def fin(r):
                for k in range(cpr):
                    sl = pl.ds(r * w_dim + k * _L, _L)
                    ms = pl.ds(k * _L, _L)
                    pa3 = ((px2_v[sl] - px1_v[sl])
                           * (py2_v[sl] - py1_v[sl]) * (1.0 / 3.0))
                    orow_v[r, ms] = jnp.where(
                        macc_v[sl] > pa3, 0.0, mrow_v[r, ms])

            for a in range(a_dim):
                pltpu.sync_copy(orow_v.at[pl.ds(a * hpw, hpw)],
                                out_hbm.at[img * a_dim + a, pl.ds(h0, hpw)])

    return sc_kernel


def kernel(batch_predict_boxes, batch_targets, no_obj_mask):
    b, a_dim, h_dim, w_dim, _ = batch_predict_boxes.shape
    pred = batch_predict_boxes.reshape(b * a_dim, h_dim, w_dim * 4)
    mask = no_obj_mask.reshape(b * a_dim, h_dim, w_dim)
    tgt = jnp.pad(batch_targets, ((0, 0), (0, _T_PAD - _T), (0, 0)))
    tgt = tgt.reshape(b, _T_PAD * 4)
    out = _make_sc_kernel(b, a_dim, h_dim, w_dim)(pred, tgt, mask)
    return out.reshape(no_obj_mask.shape)


# speedup vs baseline: 1.1748x; 1.0060x over previous
import functools

import jax
import jax.numpy as jnp
from jax import lax
from jax.experimental import pallas as pl
from jax.experimental.pallas import tpu as pltpu
from jax.experimental.pallas import tpu_sc as plsc

_NC = 2
_NS = 16
_NW = _NC * _NS
_L = 16

_T = 100
_T_PAD = 112
_TK = 4
_IN_SIZE = 512.0

_mesh = plsc.VectorSubcoreMesh(core_axis_name="c", subcore_axis_name="s")


def _make_sc_kernel(b, a_dim, h_dim, w_dim):
    hpw = h_dim // (_NW // b)
    rows = a_dim * hpw
    npw = rows * w_dim
    div = _NW // b

    @functools.partial(
        pl.kernel,
        out_type=jax.ShapeDtypeStruct((b * a_dim, h_dim, w_dim), jnp.float32),
        mesh=_mesh,
        compiler_params=pltpu.CompilerParams(needs_layout_passes=False),
        scratch_types=[
            pltpu.VMEM((rows, w_dim * 4), jnp.float32),
            pltpu.VMEM((_T_PAD * 4,), jnp.float32),
            pltpu.VMEM((rows, w_dim), jnp.float32),
            pltpu.VMEM((rows, w_dim), jnp.float32),
            pltpu.VMEM((npw,), jnp.float32),
            pltpu.VMEM((npw,), jnp.float32),
            pltpu.VMEM((npw,), jnp.float32),
            pltpu.VMEM((npw,), jnp.float32),
            pltpu.VMEM((_T_PAD,), jnp.float32),
            pltpu.VMEM((_T_PAD,), jnp.float32),
            pltpu.VMEM((_T_PAD,), jnp.float32),
            pltpu.VMEM((_T_PAD,), jnp.float32),
            pltpu.VMEM((_T_PAD,), jnp.float32),
            pltpu.VMEM((npw,), jnp.float32),
            pltpu.SemaphoreType.DMA,
            pltpu.SemaphoreType.DMA,
        ],
    )
    def sc_kernel(pred_hbm, tgt_hbm, mask_hbm, out_hbm,
                  praw_v, traw_v, mrow_v, orow_v,
                  px1_v, py1_v, px2_v, py2_v,
                  tx1_v, ty1_v, tx2_v, ty2_v, ta3_v, macc_v,
                  sem_in, sem_mask):
        wid = lax.axis_index("s") * _NC + lax.axis_index("c")
        img = wid // div
        h0 = (wid % div) * hpw

        in_copies = [
            pltpu.async_copy(pred_hbm.at[img * a_dim + a, pl.ds(h0, hpw)],
                             praw_v.at[pl.ds(a * hpw, hpw)], sem_in)
            for a in range(a_dim)
        ]
        in_copies.append(pltpu.async_copy(tgt_hbm.at[img], traw_v, sem_in))
        mask_copies = [
            pltpu.async_copy(mask_hbm.at[img * a_dim + a, pl.ds(h0, hpw)],
                             mrow_v.at[pl.ds(a * hpw, hpw)], sem_mask)
            for a in range(a_dim)
        ]
        for c in in_copies:
            c.wait()

        lane = lax.iota(jnp.int32, _L)
        lane4 = lane * 4
        cpr = w_dim // _L

        init = (jnp.full((_L,), -3.4e38, jnp.float32),
                jnp.full((_L,), 3.4e38, jnp.float32))

        @plsc.parallel_loop(0, rows, 1, unroll=2, carry=init)
        def pscan(r, ext):
            amax, amin = ext
            for k in range(cpr * 4):
                v = praw_v[r, pl.ds(k * _L, _L)]
                amax = jnp.maximum(amax, v)
                amin = jnp.minimum(amin, v)
            return amax, amin

        is_w = (lane % 4) == 2
        is_h = (lane % 4) == 3
        wmax = jnp.max(jnp.where(is_w, pscan[0], -3.4e38))
        hmax = jnp.max(jnp.where(is_h, pscan[0], -3.4e38))
        wmin = jnp.min(jnp.where(is_w, pscan[1], 3.4e38))
        hmin = jnp.min(jnp.where(is_h, pscan[1], 3.4e38))
        pamax3 = wmax * hmax * (1.0001 / 3.0)
        pamin3 = wmin * hmin * (0.9999 / 3.0)

        alive_acc = jnp.zeros((_L,), jnp.int32)
        for j in range(_T_PAD // _L):
            base = lane4 + (j * _L * 4)
            tw = plsc.load_gather(traw_v, [base + 2]) * _IN_SIZE
            th = plsc.load_gather(traw_v, [base + 3]) * _IN_SIZE
            ta3 = tw * th * (1.0 / 3.0)
            live = jnp.logical_and(ta3 < pamax3 * 6.0, ta3 * 6.0 > pamin3)
            alive_acc = alive_acc | jnp.where(live, 1, 0)
        any_alive = jnp.max(alive_acc)

        @pl.when(any_alive == 0)
        def _():
            for c in mask_copies:
                c.wait()
            outc = [
                pltpu.async_copy(mrow_v.at[pl.ds(a * hpw, hpw)],
                                 out_hbm.at[img * a_dim + a, pl.ds(h0, hpw)],
                                 sem_in)
                for a in range(a_dim)
            ]
            for c in outc:
                c.wait()

        @pl.when(any_alive > 0)
        def _():
            for c in mask_copies:
                c.wait()

            for j in range(_T_PAD // _L):
                base = lane4 + (j * _L * 4)
                cx = plsc.load_gather(traw_v, [base]) * _IN_SIZE
                cy = plsc.load_gather(traw_v, [base + 1]) * _IN_SIZE
                hw = plsc.load_gather(traw_v, [base + 2]) * (0.5 * _IN_SIZE)
                hh = plsc.load_gather(traw_v, [base + 3]) * (0.5 * _IN_SIZE)
                sl = pl.ds(j * _L, _L)
                x1 = cx - hw
                y1 = cy - hh
                x2 = cx + hw
                y2 = cy + hh
                tx1_v[sl] = x1
                ty1_v[sl] = y1
                tx2_v[sl] = x2
                ty2_v[sl] = y2
                ta3_v[sl] = (x2 - x1) * (y2 - y1) * (1.0 / 3.0)

            @plsc.parallel_loop(0, rows, 1, unroll=2)
            def pprep(r):
                rr = jnp.full((_L,), r, jnp.int32)
                for k in range(cpr):
                    base = lane4 + (k * _L * 4)
                    cx = plsc.load_gather(praw_v, [rr, base])
                    cy = plsc.load_gather(praw_v, [rr, base + 1])
                    hw = plsc.load_gather(praw_v, [rr, base + 2]) * 0.5
                    hh = plsc.load_gather(praw_v, [rr, base + 3]) * 0.5
                    sl = pl.ds(r * w_dim + k * _L, _L)
                    px1_v[sl] = cx - hw
                    py1_v[sl] = cy - hh
                    px2_v[sl] = cx + hw
                    py2_v[sl] = cy + hh
                    macc_v[sl] = jnp.zeros((_L,), jnp.float32)

            def tblk(tb, carry):
                t0 = tb * _TK
                bts = []
                for k in range(_TK):
                    idx = jnp.full((_L,), t0 + k, jnp.int32)
                    bts.append((plsc.load_gather(tx1_v, [idx]),
                                plsc.load_gather(ty1_v, [idx]),
                                plsc.load_gather(tx2_v, [idx]),
                                plsc.load_gather(ty2_v, [idx]),
                                plsc.load_gather(ta3_v, [idx])))

                blk_alive = jnp.zeros((_L,), jnp.int32)
                for (_, _, _, _, bta3) in bts:
                    live = jnp.logical_and(bta3 < pamax3 * 6.0,
                                           bta3 * 6.0 > pamin3)
                    blk_alive = blk_alive | jnp.where(live, 1, 0)

                @pl.when(jnp.max(blk_alive) > 0)
                def _():
                    @plsc.parallel_loop(0, npw // _L, 1, unroll=4)
                    def ploop(p):
                        sl = pl.ds(p * _L, _L)
                        px1 = px1_v[sl]
                        py1 = py1_v[sl]
                        px2 = px2_v[sl]
                        py2 = py2_v[sl]
                        m = macc_v[sl]
                        for (btx1, bty1, btx2, bty2, bta3) in bts:
                            iw = jnp.maximum(
                                jnp.minimum(btx2, px2) - jnp.maximum(btx1, px1),
                                0.0)
                            ih = jnp.maximum(
                                jnp.minimum(bty2, py2) - jnp.maximum(bty1, py1),
                                0.0)
                            m = jnp.maximum(m, iw * ih - bta3)
                        macc_v[sl] = m

                return carry

            lax.fori_loop(0, _T // _TK, tblk, 0)

            @plsc.parallel_loop(0, rows, 1, unroll=2)
            def fin(r):
                for k in range(cpr):
                    sl = pl.ds(r * w_dim + k * _L, _L)
                    ms = pl.ds(k * _L, _L)
                    pa3 = ((px2_v[sl] - px1_v[sl])
                           * (py2_v[sl] - py1_v[sl]) * (1.0 / 3.0))
                    orow_v[r, ms] = jnp.where(
                        macc_v[sl] > pa3, 0.0, mrow_v[r, ms])

            outc = [
                pltpu.async_copy(orow_v.at[pl.ds(a * hpw, hpw)],
                                 out_hbm.at[img * a_dim + a, pl.ds(h0, hpw)],
                                 sem_in)
                for a in range(a_dim)
            ]
            for c in outc:
                c.wait()

    return sc_kernel


def kernel(batch_predict_boxes, batch_targets, no_obj_mask):
    b, a_dim, h_dim, w_dim, _ = batch_predict_boxes.shape
    pred = batch_predict_boxes.reshape(b * a_dim, h_dim, w_dim * 4)
    mask = no_obj_mask.reshape(b * a_dim, h_dim, w_dim)
    tgt = jnp.pad(batch_targets, ((0, 0), (0, _T_PAD - _T), (0, 0)))
    tgt = tgt.reshape(b, _T_PAD * 4)
    out = _make_sc_kernel(b, a_dim, h_dim, w_dim)(pred, tgt, mask)
    return out.reshape(no_obj_mask.shape)
